# baseline (speedup 1.0000x reference)
import functools

import jax
import jax.numpy as jnp
from jax import lax
from jax.experimental import pallas as pl
from jax.experimental.pallas import tpu as pltpu
from jax.experimental.pallas import tpu_sc as plsc

_SCALE = 0.0625
_P = 7
_S = 4
_TRANS = 0.1
_N, _C, _H, _W = 2, 64, 64, 64
_R = 128
_BINS = _R * _P * _P
_NW = 16
_BPW = _BINS // _NW
_RPW = _BPW // (_P * _P)
_G = 28
_NG = _BPW // _G
_TROWS = _N * _H * _W + 192
_CNT_PAD = 400


def _body(table_hbm, rois_hbm, offx_hbm, out_hbm,
          rois_v, offx_v, idx_v, wp_v, cnt_v, rows0, rows1, outb, sem0, sem1):
    wid = lax.axis_index("s")
    pltpu.sync_copy(rois_hbm, rois_v)
    pltpu.sync_copy(offx_hbm, offx_v)

    zeros16 = jnp.zeros((16,), jnp.float32)
    for z in range(_CNT_PAD // 16):
        cnt_v[pl.ds(z * 16, 16)] = zeros16

    iot = lax.broadcasted_iota(jnp.int32, (16,), 0)
    iwf = (iot & 3).astype(jnp.float32)
    ihf = lax.shift_right_logical(iot, 2).astype(jnp.float32)
    lane_mask4 = iot < 4
    lo2 = iot & 3

    for ri in range(_RPW):
        r = wid * _RPW + ri
        rv = rois_v[pl.ds(r * 16, 16)]
        bbase = rv[0].astype(jnp.int32)
        rsw = rv[1]
        rsh = rv[2]
        roi_w = rv[3]
        roi_h = rv[4]
        bin_w = rv[5]
        bin_h = rv[6]
        sub_w = rv[7]
        sub_h = rv[8]

        def bin_body(j, _, r=r, rsw=rsw, rsh=rsh, roi_w=roi_w, roi_h=roi_h,
                     bin_w=bin_w, bin_h=bin_h, sub_w=sub_w, sub_h=sub_h,
                     bbase=bbase, ri=ri):
            ph = j // _P
            pw = j % _P
            tx = offx_v[pl.ds(r * 98 + j, 16)][0] * _TRANS
            ty = offx_v[pl.ds(r * 98 + _P * _P + j, 16)][0] * _TRANS
            wstart = pw.astype(jnp.float32) * bin_w + rsw + tx * roi_w
            hstart = ph.astype(jnp.float32) * bin_h + rsh + ty * roi_h
            w = wstart + iwf * sub_w
            h = hstart + ihf * sub_h
            valid = ((w >= -0.5) & (w <= _W - 0.5)
                     & (h >= -0.5) & (h <= _H - 0.5))
            wc = jnp.minimum(jnp.maximum(w, 0.0), float(_W - 1))
            hc = jnp.minimum(jnp.maximum(h, 0.0), float(_H - 1))
            x1 = wc.astype(jnp.int32)
            y1 = hc.astype(jnp.int32)
            dx = wc - x1.astype(jnp.float32)
            dy = hc - y1.astype(jnp.float32)
            x0 = jnp.min(x1)
            y0 = jnp.min(y1)
            vw = jnp.where(valid, 1.0, 0.0)
            omdx = 1.0 - dx
            omdy = 1.0 - dy
            i = ri * (_P * _P) + j
            ibase = i * 16
            wp_v[pl.ds(ibase, 16)] = jnp.zeros((16,), jnp.float32)
            p11 = (y1 - y0) * 4 + (x1 - x0) + ibase
            plsc.addupdate_scatter(wp_v, [p11], omdx * omdy * vw)
            plsc.addupdate_scatter(wp_v, [p11 + 1], dx * omdy * vw)
            plsc.addupdate_scatter(wp_v, [p11 + 4], omdx * dy * vw)
            plsc.addupdate_scatter(wp_v, [p11 + 5], dx * dy * vw)
            plsc.addupdate_scatter(cnt_v, [jnp.full((16,), 0, jnp.int32) + i], vw)
            base = bbase + y0 * _W + x0
            plsc.store_scatter(idx_v, [i * 4 + lo2], base + lo2 * _W,
                               mask=lane_mask4)
            return 0

        lax.fori_loop(0, _P * _P, bin_body, 0)

    bufs = (rows0, rows1)
    sems = (sem0, sem1)
    handles = [None, None]
    handles[0] = pltpu.async_copy(
        table_hbm.at[idx_v.at[pl.ds(0, _G * 4)]], bufs[0], sems[0])
    for g in range(_NG):
        if g + 1 < _NG:
            nb = (g + 1) % 2
            handles[nb] = pltpu.async_copy(
                table_hbm.at[idx_v.at[pl.ds((g + 1) * _G * 4, _G * 4)]],
                bufs[nb], sems[nb])
        handles[g % 2].wait()
        buf = bufs[g % 2]

        def red_body(jj, _, g=g, buf=buf):
            i = g * _G + jj
            ib = i * 16
            row0 = jj * 4
            wvec = wp_v[pl.ds(ib, 16)]
            c0 = cnt_v[pl.ds(i, 16)][0]
            scale = 1.0 / jnp.maximum(jnp.zeros((16,), jnp.float32) + c0, 1.0)
            acc = [jnp.zeros((16,), jnp.float32) for _ in range(4)]
            for p in range(16):
                ws = wvec[p]
                row = row0 + (p // 4)
                off = (p % 4) * 64
                for k in range(4):
                    acc[k] = acc[k] + ws * buf[row, pl.ds(off + k * 16, 16)]
            for k in range(4):
                outb[i, pl.ds(k * 16, 16)] = acc[k] * scale
            return 0

        lax.fori_loop(0, _G, red_body, 0)

    pltpu.sync_copy(outb, out_hbm.at[wid])


@jax.jit
def _scpool(table, roip, offx):
    mesh = plsc.VectorSubcoreMesh(core_axis_name="c", subcore_axis_name="s", num_cores=1)
    f = functools.partial(
        pl.kernel,
        mesh=mesh,
        compiler_params=pltpu.CompilerParams(needs_layout_passes=False),
        out_type=jax.ShapeDtypeStruct((_NW, _BPW, _C), jnp.float32),
        scratch_types=[
            pltpu.VMEM((_R * 16,), jnp.float32),
            pltpu.VMEM((_R * 2 * _P * _P + 16,), jnp.float32),
            pltpu.VMEM((_BPW * 4,), jnp.int32),
            pltpu.VMEM((_BPW * 16,), jnp.float32),
            pltpu.VMEM((_CNT_PAD,), jnp.float32),
            pltpu.VMEM((_G * 4, 4 * _C), jnp.float32),
            pltpu.VMEM((_G * 4, 4 * _C), jnp.float32),
            pltpu.VMEM((_BPW, _C), jnp.float32),
            pltpu.SemaphoreType.DMA,
            pltpu.SemaphoreType.DMA,
        ],
    )(_body)
    return f(table, roip, offx)


def kernel(input, rois, offset):
    n, c, h, w = input.shape
    flat = jnp.transpose(input, (0, 2, 3, 1)).reshape(n * h * w, c)
    pad = jnp.zeros((_TROWS + 3 - n * h * w, c), jnp.float32)
    padded = jnp.concatenate([flat, pad], axis=0)
    table = jnp.concatenate([padded[k:k + _TROWS] for k in range(4)], axis=1)
    rsw = jnp.round(rois[:, 1]) * _SCALE - 0.5
    rsh = jnp.round(rois[:, 2]) * _SCALE - 0.5
    rew = (jnp.round(rois[:, 3]) + 1.0) * _SCALE - 0.5
    reh = (jnp.round(rois[:, 4]) + 1.0) * _SCALE - 0.5
    roi_w = jnp.maximum(rew - rsw, 0.1)
    roi_h = jnp.maximum(reh - rsh, 0.1)
    bin_w = roi_w / _P
    bin_h = roi_h / _P
    sub_w = bin_w / _S
    sub_h = bin_h / _S
    bbase = rois[:, 0] * float(h * w)
    roip = jnp.stack([bbase, rsw, rsh, roi_w, roi_h, bin_w, bin_h,
                      sub_w, sub_h], axis=1)
    roip = jnp.concatenate(
        [roip, jnp.zeros((_R, 7), jnp.float32)], axis=1).reshape(-1)
    offx = offset.reshape(-1)
    offx = jnp.concatenate([offx, jnp.zeros((16,), jnp.float32)])
    out = _scpool(table, roip, offx)
    out = out.reshape(_R, _P, _P, c)
    return jnp.transpose(out, (0, 3, 1, 2))


# speedup vs baseline: 1.6145x; 1.6145x over previous
import functools

import jax
import jax.numpy as jnp
from jax import lax
from jax.experimental import pallas as pl
from jax.experimental.pallas import tpu as pltpu
from jax.experimental.pallas import tpu_sc as plsc

_SCALE = 0.0625
_P = 7
_S = 4
_TRANS = 0.1
_N, _C, _H, _W = 2, 64, 64, 64
_R = 128
_BINS = _R * _P * _P
_NW = 32
_BPW = _BINS // _NW
_RPW = _BPW // (_P * _P)
_B = 14
_NB = _BPW // _B
_TROWS = _N * _H * _W + 192
_OFFPAD = 12688
_ROIPAD = 2064


def _body(table_hbm, roip_hbm, offx_hbm, out_hbm,
          roip_v, offx_v, idx_v, wp_v, scl_v, buf0, buf1, buf2, outb,
          sem0, sem1, sem2):
    wid = lax.axis_index("s") * 2 + lax.axis_index("c")
    pltpu.sync_copy(roip_hbm, roip_v)
    pltpu.sync_copy(offx_hbm, offx_v)

    iot = lax.broadcasted_iota(jnp.int32, (16,), 0)
    m14 = iot < _B
    zeros16 = jnp.zeros((16,), jnp.float32)
    bufs = (buf0, buf1, buf2)
    sems = (sem0, sem1, sem2)

    def phase_a(blk):
        i0 = blk * _B
        bing = i0 + iot
        ibase16 = bing * 16
        rloc = lax.shift_right_logical(bing * 1339, 16)
        j16 = bing - rloc * 49
        r16 = wid * _RPW + rloc
        ph16 = lax.shift_right_logical(j16 * 9363, 16)
        pw16 = j16 - ph16 * 7
        rb = r16 * 16
        bb16 = plsc.load_gather(roip_v, [rb]).astype(jnp.int32)
        rsw = plsc.load_gather(roip_v, [rb + 1])
        rsh = plsc.load_gather(roip_v, [rb + 2])
        roi_w = plsc.load_gather(roip_v, [rb + 3])
        roi_h = plsc.load_gather(roip_v, [rb + 4])
        bin_w = plsc.load_gather(roip_v, [rb + 5])
        bin_h = plsc.load_gather(roip_v, [rb + 6])
        sub_w = plsc.load_gather(roip_v, [rb + 7])
        sub_h = plsc.load_gather(roip_v, [rb + 8])
        ob = r16 * 98 + j16
        tx = plsc.load_gather(offx_v, [ob]) * _TRANS
        ty = plsc.load_gather(offx_v, [ob + _P * _P]) * _TRANS
        wstart = pw16.astype(jnp.float32) * bin_w + rsw + tx * roi_w
        hstart = ph16.astype(jnp.float32) * bin_h + rsh + ty * roi_h
        x0 = jnp.minimum(jnp.maximum(wstart, 0.0),
                         float(_W - 1)).astype(jnp.int32)
        y0 = jnp.minimum(jnp.maximum(hstart, 0.0),
                         float(_H - 1)).astype(jnp.int32)
        base16 = bb16 + y0 * _W + x0
        for k in range(4):
            plsc.store_scatter(idx_v, [bing * 4 + k], base16 + k * _W,
                               mask=m14)
        for z in range(16):
            wp_v[pl.ds(i0 * 16 + z * 16, 16)] = zeros16
        cnt = jnp.zeros((16,), jnp.float32)
        for ih in range(_S):
            h = hstart if ih == 0 else hstart + (float(ih) * sub_h)
            validh = (h >= -0.5) & (h <= _H - 0.5)
            hc = jnp.minimum(jnp.maximum(h, 0.0), float(_H - 1))
            y1 = hc.astype(jnp.int32)
            dy = hc - y1.astype(jnp.float32)
            omdy = 1.0 - dy
            prow = ibase16 + (y1 - y0) * 4
            for iw in range(_S):
                w = wstart if iw == 0 else wstart + (float(iw) * sub_w)
                vf = jnp.where(validh & (w >= -0.5) & (w <= _W - 0.5),
                               1.0, 0.0)
                wc = jnp.minimum(jnp.maximum(w, 0.0), float(_W - 1))
                x1 = wc.astype(jnp.int32)
                dx = wc - x1.astype(jnp.float32)
                cnt = cnt + vf
                a = (1.0 - dx) * vf
                b = dx * vf
                pb = prow + (x1 - x0)
                plsc.addupdate_scatter(wp_v, [pb], a * omdy, mask=m14)
                plsc.addupdate_scatter(wp_v, [pb + 1], b * omdy, mask=m14)
                plsc.addupdate_scatter(wp_v, [pb + 4], a * dy, mask=m14)
                plsc.addupdate_scatter(wp_v, [pb + 5], b * dy, mask=m14)
        scl_v[pl.ds(i0, 16)] = 1.0 / jnp.maximum(cnt, 1.0)

    def fire(blk, t):
        pltpu.async_copy(
            table_hbm.at[idx_v.at[pl.ds(blk * (_B * 4), _B * 4)]],
            bufs[t], sems[t])

    def reduce_block(rb, t):
        buf = bufs[t]
        pltpu.make_async_copy(
            table_hbm.at[idx_v.at[pl.ds(rb * (_B * 4), _B * 4)]],
            buf, sems[t]).wait()

        def red_body(jj, _):
            i = rb * _B + jj
            wvec = wp_v[pl.ds(i * 16, 16)] * scl_v[pl.ds(i, 16)][0]
            row0 = jj * 4
            acc = [jnp.zeros((16,), jnp.float32) for _ in range(4)]
            for p in range(16):
                ws = wvec[p]
                row = row0 + (p // 4)
                off = (p % 4) * 64
                for k in range(4):
                    acc[k] = acc[k] + ws * buf[row, pl.ds(off + k * 16, 16)]
            for k in range(4):
                outb[i, pl.ds(k * 16, 16)] = acc[k]
            return 0

        lax.fori_loop(0, _B, red_body, 0)

    def mod3(v):
        return v - lax.shift_right_logical(v * 21846, 16) * 3

    def step(blk, _):
        @pl.when(blk < _NB)
        def _a():
            phase_a(blk)
            b3 = mod3(blk)
            for t in range(3):
                @pl.when(b3 == t)
                def _f(t=t):
                    fire(blk, t)

        @pl.when(blk >= 2)
        def _r():
            rb = blk - 2
            r3 = mod3(rb)
            for t in range(3):
                @pl.when(r3 == t)
                def _g(t=t):
                    reduce_block(rb, t)
        return 0

    lax.fori_loop(0, _NB + 2, step, 0)
    pltpu.sync_copy(outb, out_hbm.at[wid])


@jax.jit
def _scpool(table, roip, offx):
    mesh = plsc.VectorSubcoreMesh(core_axis_name="c", subcore_axis_name="s")
    f = functools.partial(
        pl.kernel,
        mesh=mesh,
        compiler_params=pltpu.CompilerParams(needs_layout_passes=False),
        out_type=jax.ShapeDtypeStruct((_NW, _BPW, _C), jnp.float32),
        scratch_types=[
            pltpu.VMEM((_ROIPAD,), jnp.float32),
            pltpu.VMEM((_OFFPAD,), jnp.float32),
            pltpu.VMEM((800,), jnp.int32),
            pltpu.VMEM((3344,), jnp.float32),
            pltpu.VMEM((224,), jnp.float32),
            pltpu.VMEM((_B * 4, 4 * _C), jnp.float32),
            pltpu.VMEM((_B * 4, 4 * _C), jnp.float32),
            pltpu.VMEM((_B * 4, 4 * _C), jnp.float32),
            pltpu.VMEM((_BPW, _C), jnp.float32),
            pltpu.SemaphoreType.DMA,
            pltpu.SemaphoreType.DMA,
            pltpu.SemaphoreType.DMA,
        ],
    )(_body)
    return f(table, roip, offx)


def kernel(input, rois, offset):
    n, c, h, w = input.shape
    flat = jnp.transpose(input, (0, 2, 3, 1)).reshape(n * h * w, c)
    pad = jnp.zeros((_TROWS + 3 - n * h * w, c), jnp.float32)
    padded = jnp.concatenate([flat, pad], axis=0)
    table = jnp.concatenate([padded[k:k + _TROWS] for k in range(4)], axis=1)
    rsw = jnp.round(rois[:, 1]) * _SCALE - 0.5
    rsh = jnp.round(rois[:, 2]) * _SCALE - 0.5
    rew = (jnp.round(rois[:, 3]) + 1.0) * _SCALE - 0.5
    reh = (jnp.round(rois[:, 4]) + 1.0) * _SCALE - 0.5
    roi_w = jnp.maximum(rew - rsw, 0.1)
    roi_h = jnp.maximum(reh - rsh, 0.1)
    bin_w = roi_w / _P
    bin_h = roi_h / _P
    sub_w = bin_w / _S
    sub_h = bin_h / _S
    bbase = rois[:, 0] * float(h * w)
    roip = jnp.stack([bbase, rsw, rsh, roi_w, roi_h, bin_w, bin_h,
                      sub_w, sub_h], axis=1)
    roip = jnp.concatenate(
        [roip, jnp.zeros((_R, 7), jnp.float32)], axis=1).reshape(-1)
    roip = jnp.concatenate(
        [roip, jnp.zeros((_ROIPAD - _R * 16,), jnp.float32)])
    offx = offset.reshape(-1)
    offx = jnp.concatenate(
        [offx, jnp.zeros((_OFFPAD - offx.shape[0],), jnp.float32)])
    out = _scpool(table, roip, offx)
    out = out.reshape(_R, _P, _P, c)
    return jnp.transpose(out, (0, 3, 1, 2))


# speedup vs baseline: 1.9067x; 1.1810x over previous
import functools

import jax
import jax.numpy as jnp
from jax import lax
from jax.experimental import pallas as pl
from jax.experimental.pallas import tpu as pltpu
from jax.experimental.pallas import tpu_sc as plsc

_SCALE = 0.0625
_P = 7
_S = 4
_TRANS = 0.1
_N, _C, _H, _W = 2, 64, 64, 64
_R = 128
_BINS = _R * _P * _P
_NW = 32
_BPW = _BINS // _NW
_RPW = _BPW // (_P * _P)
_B = 14
_NB = _BPW // _B
_TPAD = 8448
_OFFPAD = 12688
_ROIPAD = 2064


def _body(table_hbm, roip_hbm, offx_hbm, out_hbm,
          roip_v, offx_v, idx_v, wp_v, scl_v, buf0, buf1, buf2, outb,
          sem0, sem1, sem2):
    wid = lax.axis_index("s") * 2 + lax.axis_index("c")
    pltpu.sync_copy(roip_hbm, roip_v)
    pltpu.sync_copy(offx_hbm, offx_v)

    iot = lax.broadcasted_iota(jnp.int32, (16,), 0)
    m14 = iot < _B
    zeros16 = jnp.zeros((16,), jnp.float32)
    bufs = (buf0, buf1, buf2)
    sems = (sem0, sem1, sem2)

    def phase_a(blk):
        i0 = blk * _B
        bing = i0 + iot
        ibase16 = bing * 16
        rloc = lax.shift_right_logical(bing * 1339, 16)
        j16 = bing - rloc * 49
        r16 = wid * _RPW + rloc
        ph16 = lax.shift_right_logical(j16 * 9363, 16)
        pw16 = j16 - ph16 * 7
        rb = r16 * 16
        bb16 = plsc.load_gather(roip_v, [rb]).astype(jnp.int32)
        rsw = plsc.load_gather(roip_v, [rb + 1])
        rsh = plsc.load_gather(roip_v, [rb + 2])
        roi_w = plsc.load_gather(roip_v, [rb + 3])
        roi_h = plsc.load_gather(roip_v, [rb + 4])
        bin_w = plsc.load_gather(roip_v, [rb + 5])
        bin_h = plsc.load_gather(roip_v, [rb + 6])
        sub_w = plsc.load_gather(roip_v, [rb + 7])
        sub_h = plsc.load_gather(roip_v, [rb + 8])
        ob = r16 * 98 + j16
        tx = plsc.load_gather(offx_v, [ob]) * _TRANS
        ty = plsc.load_gather(offx_v, [ob + _P * _P]) * _TRANS
        wstart = pw16.astype(jnp.float32) * bin_w + rsw + tx * roi_w
        hstart = ph16.astype(jnp.float32) * bin_h + rsh + ty * roi_h
        x0 = jnp.minimum(jnp.maximum(wstart, 0.0),
                         float(_W - 1)).astype(jnp.int32)
        y0 = jnp.minimum(jnp.maximum(hstart, 0.0),
                         float(_H - 1)).astype(jnp.int32)
        base16 = bb16 + y0 * _W + x0
        for r in range(4):
            for half in range(2):
                plsc.store_scatter(idx_v, [bing * 8 + r * 2 + half],
                                   base16 + (r * _W + half * 2), mask=m14)
        for z in range(16):
            wp_v[pl.ds(i0 * 16 + z * 16, 16)] = zeros16
        cnt = jnp.zeros((16,), jnp.float32)
        for ih in range(_S):
            h = hstart if ih == 0 else hstart + (float(ih) * sub_h)
            validh = (h >= -0.5) & (h <= _H - 0.5)
            hc = jnp.minimum(jnp.maximum(h, 0.0), float(_H - 1))
            y1 = hc.astype(jnp.int32)
            dy = hc - y1.astype(jnp.float32)
            omdy = 1.0 - dy
            prow = ibase16 + (y1 - y0) * 4
            for iw in range(_S):
                w = wstart if iw == 0 else wstart + (float(iw) * sub_w)
                vf = jnp.where(validh & (w >= -0.5) & (w <= _W - 0.5),
                               1.0, 0.0)
                wc = jnp.minimum(jnp.maximum(w, 0.0), float(_W - 1))
                x1 = wc.astype(jnp.int32)
                dx = wc - x1.astype(jnp.float32)
                cnt = cnt + vf
                a = (1.0 - dx) * vf
                b = dx * vf
                pb = prow + (x1 - x0)
                plsc.addupdate_scatter(wp_v, [pb], a * omdy, mask=m14)
                plsc.addupdate_scatter(wp_v, [pb + 1], b * omdy, mask=m14)
                plsc.addupdate_scatter(wp_v, [pb + 4], a * dy, mask=m14)
                plsc.addupdate_scatter(wp_v, [pb + 5], b * dy, mask=m14)
        scl_v[pl.ds(i0, 16)] = 1.0 / jnp.maximum(cnt, 1.0)

    def fire(blk, t):
        pltpu.async_copy(
            table_hbm.at[idx_v.at[pl.ds(blk * (_B * 8), _B * 8)]],
            bufs[t], sems[t])

    def reduce_block(rb, t):
        buf = bufs[t]
        pltpu.make_async_copy(
            table_hbm.at[idx_v.at[pl.ds(rb * (_B * 8), _B * 8)]],
            buf, sems[t]).wait()

        def red_body(jj, _):
            i = rb * _B + jj
            wvec = wp_v[pl.ds(i * 16, 16)] * scl_v[pl.ds(i, 16)][0]
            row0 = jj * 8
            acc = [jnp.zeros((16,), jnp.float32) for _ in range(4)]
            for p in range(16):
                ws = wvec[p]
                row = row0 + (p // 4) * 2 + (p % 4) // 2
                off = (p % 2) * 64
                for k in range(4):
                    acc[k] = acc[k] + ws * buf[row, pl.ds(off + k * 16, 16)]
            for k in range(4):
                outb[i, pl.ds(k * 16, 16)] = acc[k]
            return 0

        lax.fori_loop(0, _B, red_body, 0)

    def mod3(v):
        return v - lax.shift_right_logical(v * 21846, 16) * 3

    def step(blk, _):
        @pl.when(blk < _NB)
        def _a():
            phase_a(blk)
            b3 = mod3(blk)
            for t in range(3):
                @pl.when(b3 == t)
                def _f(t=t):
                    fire(blk, t)

        @pl.when(blk >= 2)
        def _r():
            rb = blk - 2
            r3 = mod3(rb)
            for t in range(3):
                @pl.when(r3 == t)
                def _g(t=t):
                    reduce_block(rb, t)
        return 0

    lax.fori_loop(0, _NB + 2, step, 0)
    pltpu.sync_copy(outb, out_hbm.at[wid])


@jax.jit
def _scpool(table, roip, offx):
    mesh = plsc.VectorSubcoreMesh(core_axis_name="c", subcore_axis_name="s")
    f = functools.partial(
        pl.kernel,
        mesh=mesh,
        compiler_params=pltpu.CompilerParams(needs_layout_passes=False),
        out_type=jax.ShapeDtypeStruct((_NW, _BPW, _C), jnp.float32),
        scratch_types=[
            pltpu.VMEM((_ROIPAD,), jnp.float32),
            pltpu.VMEM((_OFFPAD,), jnp.float32),
            pltpu.VMEM((1600,), jnp.int32),
            pltpu.VMEM((3344,), jnp.float32),
            pltpu.VMEM((224,), jnp.float32),
            pltpu.VMEM((_B * 8, 2 * _C), jnp.float32),
            pltpu.VMEM((_B * 8, 2 * _C), jnp.float32),
            pltpu.VMEM((_B * 8, 2 * _C), jnp.float32),
            pltpu.VMEM((_BPW, _C), jnp.float32),
            pltpu.SemaphoreType.DMA,
            pltpu.SemaphoreType.DMA,
            pltpu.SemaphoreType.DMA,
        ],
    )(_body)
    return f(table, roip, offx)


def kernel(input, rois, offset):
    n, c, h, w = input.shape
    flat = jnp.transpose(input, (0, 2, 3, 1)).reshape(n * h * w, c)
    flatp = jnp.concatenate(
        [flat, jnp.zeros((_TPAD + 1 - n * h * w, c), jnp.float32)], axis=0)
    table = jnp.concatenate([flatp[:-1], flatp[1:]], axis=1)
    rsw = jnp.round(rois[:, 1]) * _SCALE - 0.5
    rsh = jnp.round(rois[:, 2]) * _SCALE - 0.5
    rew = (jnp.round(rois[:, 3]) + 1.0) * _SCALE - 0.5
    reh = (jnp.round(rois[:, 4]) + 1.0) * _SCALE - 0.5
    roi_w = jnp.maximum(rew - rsw, 0.1)
    roi_h = jnp.maximum(reh - rsh, 0.1)
    bin_w = roi_w / _P
    bin_h = roi_h / _P
    sub_w = bin_w / _S
    sub_h = bin_h / _S
    bbase = rois[:, 0] * float(h * w)
    roip = jnp.stack([bbase, rsw, rsh, roi_w, roi_h, bin_w, bin_h,
                      sub_w, sub_h], axis=1)
    roip = jnp.concatenate(
        [roip, jnp.zeros((_R, 7), jnp.float32)], axis=1).reshape(-1)
    roip = jnp.concatenate(
        [roip, jnp.zeros((_ROIPAD - _R * 16,), jnp.float32)])
    offx = offset.reshape(-1)
    offx = jnp.concatenate(
        [offx, jnp.zeros((_OFFPAD - offx.shape[0],), jnp.float32)])
    out = _scpool(table, roip, offx)
    out = out.reshape(_R, _P, _P, c)
    return jnp.transpose(out, (0, 3, 1, 2))
